# Initial kernel scaffold; baseline (speedup 1.0000x reference)
import jax
import jax.numpy as jnp
from jax.experimental import pallas as pl


def kernel(x, edge_index_rel0, edge_index_rel1, W_self1_0, W_neigh1_0, b1_0, W_self1_1, W_neigh1_1, b1_1, W_self2_0, W_neigh2_0, b2_0, W_self2_1, W_neigh2_1, b2_1):
    raise NotImplementedError("write your pallas kernel here")


# speedup vs baseline: 4.1236x; 4.1236x over previous
import functools

import jax
import jax.numpy as jnp
from jax import lax
from jax.experimental import pallas as pl
from jax.experimental.pallas import tpu as pltpu
from jax.experimental.pallas import tpu_sc as plsc

N = 10000
NPAD = 10240
E = 320000
D = 128
NSUB = 16
ROWS_PER_TILE = NPAD // NSUB
EDGES_PER_TILE = E // NSUB
CHUNK = 80
NCHUNK = EDGES_PER_TILE // CHUNK
N_HOP = ROWS_PER_TILE // CHUNK
DEG_W = 16


def _sc_agg_body(x_hbm, edges_hbm, zacc_hbm, zdeg_hbm,
                 p_hbm, dpart_hbm,
                 acc, src_v, dst_v, rows_v, degacc_v, sem):
    c = lax.axis_index("c")
    s = lax.axis_index("s")

    row0 = s * ROWS_PER_TILE
    pltpu.sync_copy(zacc_hbm, rows_v)
    pltpu.sync_copy(zdeg_hbm, degacc_v)
    for j in range(N_HOP):
        pltpu.sync_copy(rows_v, acc.at[pl.ds(row0 + j * CHUNK, CHUNK)])
    plsc.subcore_barrier()

    base = s * EDGES_PER_TILE
    rel0 = c * (2 * E)

    def step(i, carry):
        off = base + i * CHUNK
        pltpu.sync_copy(edges_hbm.at[pl.ds(rel0 + off, CHUNK)], src_v)
        pltpu.sync_copy(edges_hbm.at[pl.ds(rel0 + E + off, CHUNK)], dst_v)
        pltpu.async_copy(x_hbm.at[src_v], rows_v, sem).wait()
        pltpu.sync_copy(rows_v, acc.at[dst_v], add=True)
        for k in range(CHUNK // 16):
            idx16 = dst_v[pl.ds(k * 16, 16)]
            plsc.addupdate_scatter(degacc_v, [idx16],
                                   jnp.full((16,), 1.0, jnp.float32))
        return carry

    lax.fori_loop(0, NCHUNK, step, 0)
    pltpu.sync_copy(degacc_v,
                    dpart_hbm.at[pl.ds((c * NSUB + s) * NPAD, NPAD)])
    plsc.subcore_barrier()

    out0 = c * NPAD + row0
    for j in range(N_HOP):
        pltpu.sync_copy(acc.at[pl.ds(row0 + j * CHUNK, CHUNK)], rows_v)
        pltpu.sync_copy(rows_v, p_hbm.at[pl.ds(out0 + j * CHUNK, CHUNK)])


_sc_agg = pl.kernel(
    _sc_agg_body,
    out_type=(
        jax.ShapeDtypeStruct((2 * NPAD, D), jnp.float32),
        jax.ShapeDtypeStruct((2 * NSUB * NPAD,), jnp.float32),
    ),
    mesh=plsc.VectorSubcoreMesh(core_axis_name="c", subcore_axis_name="s",
                                num_cores=2, num_subcores=NSUB),
    compiler_params=pltpu.CompilerParams(needs_layout_passes=False),
    scratch_types=[
        pltpu.VMEM_SHARED((NPAD, D), jnp.float32),
        pltpu.VMEM((CHUNK,), jnp.int32),
        pltpu.VMEM((CHUNK,), jnp.int32),
        pltpu.VMEM((CHUNK, D), jnp.float32),
        pltpu.VMEM((NPAD,), jnp.float32),
        pltpu.SemaphoreType.DMA,
    ],
)


def _tc_layer_body(x_ref, p0_ref, p1_ref, d0_ref, d1_ref,
                   ws0_ref, ws1_ref, wn0_ref, wn1_ref, b0_ref, b1_ref,
                   out_ref, *, relu):
    ones16 = jnp.ones((NSUB, 1), jnp.float32)
    dn = (((0,), (0,)), ((), ()))
    d0 = lax.dot_general(d0_ref[...], ones16, dn,
                         preferred_element_type=jnp.float32)
    d1 = lax.dot_general(d1_ref[...], ones16, dn,
                         preferred_element_type=jnp.float32)
    r0 = 1.0 / jnp.maximum(d0, 1.0)
    r1 = 1.0 / jnp.maximum(d1, 1.0)
    ws = ws0_ref[...] + ws1_ref[...]
    acc = jnp.dot(x_ref[...], ws, preferred_element_type=jnp.float32)
    acc += jnp.dot(p0_ref[...] * r0, wn0_ref[...],
                   preferred_element_type=jnp.float32)
    acc += jnp.dot(p1_ref[...] * r1, wn1_ref[...],
                   preferred_element_type=jnp.float32)
    acc += b0_ref[...] + b1_ref[...]
    out_ref[...] = jnp.maximum(acc, 0.0) if relu else acc


def _tc_layer(relu, x, p0, p1, d0, d1, ws0, ws1, wn0, wn1, b0, b1):
    blk = 1024
    grid = (NPAD // blk,)
    row_spec = pl.BlockSpec((blk, D), lambda i: (i, 0))
    deg_spec = pl.BlockSpec((NSUB, blk), lambda i: (0, i))
    w_spec = pl.BlockSpec((D, D), lambda i: (0, 0))
    b_spec = pl.BlockSpec((1, D), lambda i: (0, 0))
    return pl.pallas_call(
        functools.partial(_tc_layer_body, relu=relu),
        grid=grid,
        in_specs=[row_spec, row_spec, row_spec, deg_spec, deg_spec,
                  w_spec, w_spec, w_spec, w_spec, b_spec, b_spec],
        out_specs=row_spec,
        out_shape=jax.ShapeDtypeStruct((NPAD, D), jnp.float32),
    )(x, p0, p1, d0, d1, ws0, ws1, wn0, wn1, b0, b1)


def kernel(x, edge_index_rel0, edge_index_rel1,
           W_self1_0, W_neigh1_0, b1_0, W_self1_1, W_neigh1_1, b1_1,
           W_self2_0, W_neigh2_0, b2_0, W_self2_1, W_neigh2_1, b2_1):
    edges = jnp.concatenate([edge_index_rel0.reshape(-1),
                             edge_index_rel1.reshape(-1)])
    zacc = jnp.zeros((CHUNK, D), jnp.float32)
    zdeg = jnp.zeros((NPAD,), jnp.float32)
    xp = jnp.pad(x, ((0, NPAD - N), (0, 0)))

    p1, dpart = _sc_agg(xp, edges, zacc, zdeg)
    p1 = p1.reshape(2, NPAD, D)
    dp = dpart.reshape(2 * NSUB, NPAD)
    d0, d1 = dp[:NSUB], dp[NSUB:]
    h = _tc_layer(True, xp, p1[0], p1[1], d0, d1,
                  W_self1_0, W_self1_1, W_neigh1_0, W_neigh1_1,
                  b1_0.reshape(1, D), b1_1.reshape(1, D))
    p2, _ = _sc_agg(h, edges, zacc, zdeg)
    p2 = p2.reshape(2, NPAD, D)
    out = _tc_layer(False, h, p2[0], p2[1], d0, d1,
                    W_self2_0, W_self2_1, W_neigh2_0, W_neigh2_1,
                    b2_0.reshape(1, D), b2_1.reshape(1, D))
    return out[:N]


# speedup vs baseline: 4.9548x; 1.2016x over previous
import functools

import jax
import jax.numpy as jnp
from jax import lax
from jax.experimental import pallas as pl
from jax.experimental.pallas import tpu as pltpu
from jax.experimental.pallas import tpu_sc as plsc

N = 10000
NPAD = 10240
E = 320000
D = 128
NSUB = 16
ROWS_PER_TILE = NPAD // NSUB
EDGES_PER_TILE = E // NSUB
CHUNK = 80
NCHUNK = EDGES_PER_TILE // CHUNK
N_HOP = ROWS_PER_TILE // CHUNK
DEG_W = 16


def _sc_agg_body(x_hbm, edges_hbm, zacc_hbm, zdeg_hbm,
                 p_hbm, dpart_hbm,
                 acc, src_v, dst_v, rows3, degacc_v, sem):
    c = lax.axis_index("c")
    s = lax.axis_index("s")

    row0 = s * ROWS_PER_TILE
    pltpu.sync_copy(zacc_hbm, rows3.at[0])
    pltpu.sync_copy(zdeg_hbm, degacc_v)
    for j in range(N_HOP):
        pltpu.sync_copy(rows3.at[0], acc.at[pl.ds(row0 + j * CHUNK, CHUNK)])
    plsc.subcore_barrier()

    base = s * EDGES_PER_TILE
    rel0 = c * (2 * E)

    pltpu.sync_copy(edges_hbm.at[pl.ds(rel0 + base, CHUNK)], src_v)
    pltpu.async_copy(x_hbm.at[src_v], rows3.at[0], sem)

    def step(i, carry):
        p = lax.rem(i, 2)
        q = 1 - p
        off = base + i * CHUNK
        pltpu.make_async_copy(x_hbm.at[src_v], rows3.at[p], sem).wait()
        pltpu.sync_copy(edges_hbm.at[pl.ds(rel0 + E + off, CHUNK)], dst_v)

        @pl.when(i < NCHUNK - 1)
        def _():
            pltpu.sync_copy(edges_hbm.at[pl.ds(rel0 + off + CHUNK, CHUNK)],
                            src_v)
            pltpu.async_copy(x_hbm.at[src_v], rows3.at[q], sem)
        for k in range(CHUNK // 16):
            idx16 = dst_v[pl.ds(k * 16, 16)]
            plsc.addupdate_scatter(degacc_v, [idx16],
                                   jnp.full((16,), 1.0, jnp.float32))
        pltpu.sync_copy(rows3.at[p], acc.at[dst_v], add=True)
        return carry

    lax.fori_loop(0, NCHUNK, step, 0)
    pltpu.sync_copy(degacc_v,
                    dpart_hbm.at[pl.ds((c * NSUB + s) * NPAD, NPAD)])
    plsc.subcore_barrier()

    out0 = c * NPAD + row0
    for j in range(N_HOP):
        pltpu.sync_copy(acc.at[pl.ds(row0 + j * CHUNK, CHUNK)], rows3.at[0])
        pltpu.sync_copy(rows3.at[0], p_hbm.at[pl.ds(out0 + j * CHUNK, CHUNK)])


_sc_agg = pl.kernel(
    _sc_agg_body,
    out_type=(
        jax.ShapeDtypeStruct((2 * NPAD, D), jnp.float32),
        jax.ShapeDtypeStruct((2 * NSUB * NPAD,), jnp.float32),
    ),
    mesh=plsc.VectorSubcoreMesh(core_axis_name="c", subcore_axis_name="s",
                                num_cores=2, num_subcores=NSUB),
    compiler_params=pltpu.CompilerParams(needs_layout_passes=False),
    scratch_types=[
        pltpu.VMEM_SHARED((NPAD, D), jnp.float32),
        pltpu.VMEM((CHUNK,), jnp.int32),
        pltpu.VMEM((CHUNK,), jnp.int32),
        pltpu.VMEM((2, CHUNK, D), jnp.float32),
        pltpu.VMEM((NPAD,), jnp.float32),
        pltpu.SemaphoreType.DMA,
    ],
)


def _tc_layer_body(x_ref, p0_ref, p1_ref, d0_ref, d1_ref,
                   ws0_ref, ws1_ref, wn0_ref, wn1_ref, b0_ref, b1_ref,
                   out_ref, *, relu):
    ones16 = jnp.ones((NSUB, 1), jnp.float32)
    dn = (((0,), (0,)), ((), ()))
    d0 = lax.dot_general(d0_ref[...], ones16, dn,
                         preferred_element_type=jnp.float32)
    d1 = lax.dot_general(d1_ref[...], ones16, dn,
                         preferred_element_type=jnp.float32)
    r0 = 1.0 / jnp.maximum(d0, 1.0)
    r1 = 1.0 / jnp.maximum(d1, 1.0)
    ws = ws0_ref[...] + ws1_ref[...]
    acc = jnp.dot(x_ref[...], ws, preferred_element_type=jnp.float32)
    acc += jnp.dot(p0_ref[...] * r0, wn0_ref[...],
                   preferred_element_type=jnp.float32)
    acc += jnp.dot(p1_ref[...] * r1, wn1_ref[...],
                   preferred_element_type=jnp.float32)
    acc += b0_ref[...] + b1_ref[...]
    out_ref[...] = jnp.maximum(acc, 0.0) if relu else acc


def _tc_layer(relu, x, p0, p1, d0, d1, ws0, ws1, wn0, wn1, b0, b1):
    blk = 1024
    grid = (NPAD // blk,)
    row_spec = pl.BlockSpec((blk, D), lambda i: (i, 0))
    deg_spec = pl.BlockSpec((NSUB, blk), lambda i: (0, i))
    w_spec = pl.BlockSpec((D, D), lambda i: (0, 0))
    b_spec = pl.BlockSpec((1, D), lambda i: (0, 0))
    return pl.pallas_call(
        functools.partial(_tc_layer_body, relu=relu),
        grid=grid,
        in_specs=[row_spec, row_spec, row_spec, deg_spec, deg_spec,
                  w_spec, w_spec, w_spec, w_spec, b_spec, b_spec],
        out_specs=row_spec,
        out_shape=jax.ShapeDtypeStruct((NPAD, D), jnp.float32),
    )(x, p0, p1, d0, d1, ws0, ws1, wn0, wn1, b0, b1)


def kernel(x, edge_index_rel0, edge_index_rel1,
           W_self1_0, W_neigh1_0, b1_0, W_self1_1, W_neigh1_1, b1_1,
           W_self2_0, W_neigh2_0, b2_0, W_self2_1, W_neigh2_1, b2_1):
    edges = jnp.concatenate([edge_index_rel0.reshape(-1),
                             edge_index_rel1.reshape(-1)])
    zacc = jnp.zeros((CHUNK, D), jnp.float32)
    zdeg = jnp.zeros((NPAD,), jnp.float32)
    xp = jnp.pad(x, ((0, NPAD - N), (0, 0)))

    p1, dpart = _sc_agg(xp, edges, zacc, zdeg)
    p1 = p1.reshape(2, NPAD, D)
    dp = dpart.reshape(2 * NSUB, NPAD)
    d0, d1 = dp[:NSUB], dp[NSUB:]
    h = _tc_layer(True, xp, p1[0], p1[1], d0, d1,
                  W_self1_0, W_self1_1, W_neigh1_0, W_neigh1_1,
                  b1_0.reshape(1, D), b1_1.reshape(1, D))
    p2, _ = _sc_agg(h, edges, zacc, zdeg)
    p2 = p2.reshape(2, NPAD, D)
    out = _tc_layer(False, h, p2[0], p2[1], d0, d1,
                    W_self2_0, W_self2_1, W_neigh2_0, W_neigh2_1,
                    b2_0.reshape(1, D), b2_1.reshape(1, D))
    return out[:N]


# speedup vs baseline: 7.9170x; 1.5978x over previous
import functools

import jax
import jax.numpy as jnp
from jax import lax
from jax.experimental import pallas as pl
from jax.experimental.pallas import tpu as pltpu
from jax.experimental.pallas import tpu_sc as plsc

N = 10000
NPAD = 10240
E = 320000
D = 128
NSUB = 16
ROWS_PER_TILE = NPAD // NSUB
EDGES_PER_TILE = E // NSUB
CHUNK = 80
NCHUNK = EDGES_PER_TILE // CHUNK
N_HOP = ROWS_PER_TILE // CHUNK
DEG_W = 16


def _sc_agg_body(x_hbm, edges_hbm, zacc_hbm, zdeg_hbm,
                 p_hbm, dpart_hbm,
                 acc, sp, dp, rows3, degacc_v, gs, si, di):
    c = lax.axis_index("c")
    s = lax.axis_index("s")

    row0 = s * ROWS_PER_TILE
    pltpu.sync_copy(zacc_hbm, rows3.at[0])
    pltpu.sync_copy(zdeg_hbm, degacc_v)
    for j in range(N_HOP):
        pltpu.sync_copy(rows3.at[0], acc.at[pl.ds(row0 + j * CHUNK, CHUNK)])
    plsc.subcore_barrier()

    base = s * EDGES_PER_TILE
    rel0 = c * (2 * E)

    def src_off(i):
        return rel0 + base + i * CHUNK

    def dst_off(i):
        return rel0 + E + base + i * CHUNK

    pltpu.sync_copy(edges_hbm.at[pl.ds(src_off(0), CHUNK)], sp.at[0])
    pltpu.sync_copy(edges_hbm.at[pl.ds(dst_off(0), CHUNK)], dp.at[0])
    pltpu.async_copy(x_hbm.at[sp.at[0]], rows3.at[0], gs)
    pltpu.async_copy(edges_hbm.at[pl.ds(src_off(1), CHUNK)], sp.at[1], si)
    pltpu.async_copy(edges_hbm.at[pl.ds(dst_off(1), CHUNK)], dp.at[1], di)

    def step(i, carry):
        p = lax.rem(i, 2)
        q = 1 - p
        pltpu.make_async_copy(x_hbm.at[sp.at[p]], rows3.at[p], gs).wait()

        @pl.when(i < NCHUNK - 1)
        def _():
            pltpu.make_async_copy(
                edges_hbm.at[pl.ds(src_off(i + 1), CHUNK)], sp.at[q],
                si).wait()
            pltpu.async_copy(x_hbm.at[sp.at[q]], rows3.at[q], gs)

        @pl.when(i < NCHUNK - 2)
        def _():
            pltpu.async_copy(
                edges_hbm.at[pl.ds(src_off(i + 2), CHUNK)], sp.at[p], si)

        @pl.when(i > 0)
        def _():
            pltpu.make_async_copy(
                edges_hbm.at[pl.ds(dst_off(i), CHUNK)], dp.at[p], di).wait()

        @pl.when(i < NCHUNK - 1)
        def _():
            pltpu.async_copy(
                edges_hbm.at[pl.ds(dst_off(i + 1), CHUNK)], dp.at[q], di)
        for k in range(CHUNK // 16):
            idx16 = dp[p, pl.ds(k * 16, 16)]
            plsc.addupdate_scatter(degacc_v, [idx16],
                                   jnp.full((16,), 1.0, jnp.float32))
        pltpu.sync_copy(rows3.at[p], acc.at[dp.at[p]], add=True)
        return carry

    lax.fori_loop(0, NCHUNK, step, 0)
    pltpu.sync_copy(degacc_v,
                    dpart_hbm.at[pl.ds((c * NSUB + s) * NPAD, NPAD)])
    plsc.subcore_barrier()

    out0 = c * NPAD + row0
    for j in range(N_HOP):
        pltpu.sync_copy(acc.at[pl.ds(row0 + j * CHUNK, CHUNK)], rows3.at[0])
        pltpu.sync_copy(rows3.at[0], p_hbm.at[pl.ds(out0 + j * CHUNK, CHUNK)])


_sc_agg = pl.kernel(
    _sc_agg_body,
    out_type=(
        jax.ShapeDtypeStruct((2 * NPAD, D), jnp.float32),
        jax.ShapeDtypeStruct((2 * NSUB * NPAD,), jnp.float32),
    ),
    mesh=plsc.VectorSubcoreMesh(core_axis_name="c", subcore_axis_name="s",
                                num_cores=2, num_subcores=NSUB),
    compiler_params=pltpu.CompilerParams(needs_layout_passes=False),
    scratch_types=[
        pltpu.VMEM_SHARED((NPAD, D), jnp.float32),
        pltpu.VMEM((2, CHUNK), jnp.int32),
        pltpu.VMEM((2, CHUNK), jnp.int32),
        pltpu.VMEM((2, CHUNK, D), jnp.float32),
        pltpu.VMEM((NPAD,), jnp.float32),
        pltpu.SemaphoreType.DMA,
        pltpu.SemaphoreType.DMA,
        pltpu.SemaphoreType.DMA,
    ],
)


def _tc_layer_body(x_ref, p0_ref, p1_ref, d0_ref, d1_ref,
                   ws0_ref, ws1_ref, wn0_ref, wn1_ref, b0_ref, b1_ref,
                   out_ref, *, relu):
    ones16 = jnp.ones((NSUB, 1), jnp.float32)
    dn = (((0,), (0,)), ((), ()))
    d0 = lax.dot_general(d0_ref[...], ones16, dn,
                         preferred_element_type=jnp.float32)
    d1 = lax.dot_general(d1_ref[...], ones16, dn,
                         preferred_element_type=jnp.float32)
    r0 = 1.0 / jnp.maximum(d0, 1.0)
    r1 = 1.0 / jnp.maximum(d1, 1.0)
    ws = ws0_ref[...] + ws1_ref[...]
    acc = jnp.dot(x_ref[...], ws, preferred_element_type=jnp.float32)
    acc += jnp.dot(p0_ref[...] * r0, wn0_ref[...],
                   preferred_element_type=jnp.float32)
    acc += jnp.dot(p1_ref[...] * r1, wn1_ref[...],
                   preferred_element_type=jnp.float32)
    acc += b0_ref[...] + b1_ref[...]
    out_ref[...] = jnp.maximum(acc, 0.0) if relu else acc


def _tc_layer(relu, x, p0, p1, d0, d1, ws0, ws1, wn0, wn1, b0, b1):
    blk = 1024
    grid = (NPAD // blk,)
    row_spec = pl.BlockSpec((blk, D), lambda i: (i, 0))
    deg_spec = pl.BlockSpec((NSUB, blk), lambda i: (0, i))
    w_spec = pl.BlockSpec((D, D), lambda i: (0, 0))
    b_spec = pl.BlockSpec((1, D), lambda i: (0, 0))
    return pl.pallas_call(
        functools.partial(_tc_layer_body, relu=relu),
        grid=grid,
        in_specs=[row_spec, row_spec, row_spec, deg_spec, deg_spec,
                  w_spec, w_spec, w_spec, w_spec, b_spec, b_spec],
        out_specs=row_spec,
        out_shape=jax.ShapeDtypeStruct((NPAD, D), jnp.float32),
    )(x, p0, p1, d0, d1, ws0, ws1, wn0, wn1, b0, b1)


def kernel(x, edge_index_rel0, edge_index_rel1,
           W_self1_0, W_neigh1_0, b1_0, W_self1_1, W_neigh1_1, b1_1,
           W_self2_0, W_neigh2_0, b2_0, W_self2_1, W_neigh2_1, b2_1):
    edges = jnp.concatenate([edge_index_rel0.reshape(-1),
                             edge_index_rel1.reshape(-1)])
    zacc = jnp.zeros((CHUNK, D), jnp.float32)
    zdeg = jnp.zeros((NPAD,), jnp.float32)
    xp = jnp.pad(x, ((0, NPAD - N), (0, 0)))

    p1, dpart = _sc_agg(xp, edges, zacc, zdeg)
    p1 = p1.reshape(2, NPAD, D)
    dp = dpart.reshape(2 * NSUB, NPAD)
    d0, d1 = dp[:NSUB], dp[NSUB:]
    h = _tc_layer(True, xp, p1[0], p1[1], d0, d1,
                  W_self1_0, W_self1_1, W_neigh1_0, W_neigh1_1,
                  b1_0.reshape(1, D), b1_1.reshape(1, D))
    p2, _ = _sc_agg(h, edges, zacc, zdeg)
    p2 = p2.reshape(2, NPAD, D)
    out = _tc_layer(False, h, p2[0], p2[1], d0, d1,
                    W_self2_0, W_self2_1, W_neigh2_0, W_neigh2_1,
                    b2_0.reshape(1, D), b2_1.reshape(1, D))
    return out[:N]


# speedup vs baseline: 7.9252x; 1.0010x over previous
import functools

import jax
import jax.numpy as jnp
from jax import lax
from jax.experimental import pallas as pl
from jax.experimental.pallas import tpu as pltpu
from jax.experimental.pallas import tpu_sc as plsc

N = 10000
NPAD = 10240
E = 320000
D = 128
NSUB = 16
ROWS_PER_TILE = NPAD // NSUB
EDGES_PER_TILE = E // NSUB
CHUNK = 80
NCHUNK = EDGES_PER_TILE // CHUNK
N_HOP = ROWS_PER_TILE // CHUNK
DEG_W = 16


def _sc_agg_body(x_hbm, edges_hbm, zacc_hbm, zdeg_hbm,
                 p_hbm, dpart_hbm,
                 acc, sp, dp, dst_v, rows3, degacc_v, gs, si, di):
    c = lax.axis_index("c")
    s = lax.axis_index("s")

    row0 = s * ROWS_PER_TILE
    pltpu.sync_copy(zacc_hbm, rows3.at[0])
    pltpu.sync_copy(zdeg_hbm, degacc_v)
    for j in range(N_HOP):
        pltpu.sync_copy(rows3.at[0], acc.at[pl.ds(row0 + j * CHUNK, CHUNK)])
    plsc.subcore_barrier()

    base = s * EDGES_PER_TILE
    rel0 = c * (2 * E)

    def src_off(i):
        return rel0 + base + i * CHUNK

    def dst_off(i):
        return rel0 + E + base + i * CHUNK

    pltpu.sync_copy(edges_hbm.at[pl.ds(src_off(0), CHUNK)], sp.at[0])
    pltpu.sync_copy(edges_hbm.at[pl.ds(dst_off(0), CHUNK)], dp.at[0])
    pltpu.async_copy(x_hbm.at[sp.at[0]], rows3.at[0], gs)
    pltpu.async_copy(edges_hbm.at[pl.ds(src_off(1), CHUNK)], sp.at[1], si)
    pltpu.async_copy(edges_hbm.at[pl.ds(dst_off(1), CHUNK)], dp.at[1], di)

    def step(i, carry):
        p = lax.rem(i, 2)
        q = 1 - p
        pltpu.make_async_copy(x_hbm.at[sp.at[p]], rows3.at[p], gs).wait()

        @pl.when(i < NCHUNK - 1)
        def _():
            pltpu.make_async_copy(
                edges_hbm.at[pl.ds(src_off(i + 1), CHUNK)], sp.at[q],
                si).wait()
            pltpu.async_copy(x_hbm.at[sp.at[q]], rows3.at[q], gs)

        @pl.when(i < NCHUNK - 2)
        def _():
            pltpu.async_copy(
                edges_hbm.at[pl.ds(src_off(i + 2), CHUNK)], sp.at[p], si)

        @pl.when(i > 0)
        def _():
            pltpu.make_async_copy(
                edges_hbm.at[pl.ds(dst_off(i), CHUNK)], dp.at[p], di).wait()

        @pl.when(i < NCHUNK - 1)
        def _():
            pltpu.async_copy(
                edges_hbm.at[pl.ds(dst_off(i + 1), CHUNK)], dp.at[q], di)
        for k in range(CHUNK // 16):
            idx16 = dp[p, pl.ds(k * 16, 16)]
            dst_v[pl.ds(k * 16, 16)] = idx16
            plsc.addupdate_scatter(degacc_v, [idx16],
                                   jnp.full((16,), 1.0, jnp.float32))
        pltpu.sync_copy(rows3.at[p], acc.at[dst_v], add=True)
        return carry

    lax.fori_loop(0, NCHUNK, step, 0)
    pltpu.sync_copy(degacc_v,
                    dpart_hbm.at[pl.ds((c * NSUB + s) * NPAD, NPAD)])
    plsc.subcore_barrier()

    out0 = c * NPAD + row0
    for j in range(N_HOP):
        pltpu.sync_copy(acc.at[pl.ds(row0 + j * CHUNK, CHUNK)], rows3.at[0])
        pltpu.sync_copy(rows3.at[0], p_hbm.at[pl.ds(out0 + j * CHUNK, CHUNK)])


_sc_agg = pl.kernel(
    _sc_agg_body,
    out_type=(
        jax.ShapeDtypeStruct((2 * NPAD, D), jnp.float32),
        jax.ShapeDtypeStruct((2 * NSUB * NPAD,), jnp.float32),
    ),
    mesh=plsc.VectorSubcoreMesh(core_axis_name="c", subcore_axis_name="s",
                                num_cores=2, num_subcores=NSUB),
    compiler_params=pltpu.CompilerParams(needs_layout_passes=False),
    scratch_types=[
        pltpu.VMEM_SHARED((NPAD, D), jnp.float32),
        pltpu.VMEM((2, CHUNK), jnp.int32),
        pltpu.VMEM((2, CHUNK), jnp.int32),
        pltpu.VMEM((CHUNK,), jnp.int32),
        pltpu.VMEM((2, CHUNK, D), jnp.float32),
        pltpu.VMEM((NPAD,), jnp.float32),
        pltpu.SemaphoreType.DMA,
        pltpu.SemaphoreType.DMA,
        pltpu.SemaphoreType.DMA,
    ],
)


def _tc_layer_body(x_ref, p0_ref, p1_ref, d0_ref, d1_ref,
                   ws0_ref, ws1_ref, wn0_ref, wn1_ref, b0_ref, b1_ref,
                   out_ref, *, relu):
    ones16 = jnp.ones((NSUB, 1), jnp.float32)
    dn = (((0,), (0,)), ((), ()))
    d0 = lax.dot_general(d0_ref[...], ones16, dn,
                         preferred_element_type=jnp.float32)
    d1 = lax.dot_general(d1_ref[...], ones16, dn,
                         preferred_element_type=jnp.float32)
    r0 = 1.0 / jnp.maximum(d0, 1.0)
    r1 = 1.0 / jnp.maximum(d1, 1.0)
    ws = ws0_ref[...] + ws1_ref[...]
    acc = jnp.dot(x_ref[...], ws, preferred_element_type=jnp.float32)
    acc += jnp.dot(p0_ref[...] * r0, wn0_ref[...],
                   preferred_element_type=jnp.float32)
    acc += jnp.dot(p1_ref[...] * r1, wn1_ref[...],
                   preferred_element_type=jnp.float32)
    acc += b0_ref[...] + b1_ref[...]
    out_ref[...] = jnp.maximum(acc, 0.0) if relu else acc


def _tc_layer(relu, x, p0, p1, d0, d1, ws0, ws1, wn0, wn1, b0, b1):
    blk = 1024
    grid = (NPAD // blk,)
    row_spec = pl.BlockSpec((blk, D), lambda i: (i, 0))
    deg_spec = pl.BlockSpec((NSUB, blk), lambda i: (0, i))
    w_spec = pl.BlockSpec((D, D), lambda i: (0, 0))
    b_spec = pl.BlockSpec((1, D), lambda i: (0, 0))
    return pl.pallas_call(
        functools.partial(_tc_layer_body, relu=relu),
        grid=grid,
        in_specs=[row_spec, row_spec, row_spec, deg_spec, deg_spec,
                  w_spec, w_spec, w_spec, w_spec, b_spec, b_spec],
        out_specs=row_spec,
        out_shape=jax.ShapeDtypeStruct((NPAD, D), jnp.float32),
    )(x, p0, p1, d0, d1, ws0, ws1, wn0, wn1, b0, b1)


def kernel(x, edge_index_rel0, edge_index_rel1,
           W_self1_0, W_neigh1_0, b1_0, W_self1_1, W_neigh1_1, b1_1,
           W_self2_0, W_neigh2_0, b2_0, W_self2_1, W_neigh2_1, b2_1):
    edges = jnp.concatenate([edge_index_rel0.reshape(-1),
                             edge_index_rel1.reshape(-1)])
    zacc = jnp.zeros((CHUNK, D), jnp.float32)
    zdeg = jnp.zeros((NPAD,), jnp.float32)
    xp = jnp.pad(x, ((0, NPAD - N), (0, 0)))

    p1, dpart = _sc_agg(xp, edges, zacc, zdeg)
    p1 = p1.reshape(2, NPAD, D)
    dp = dpart.reshape(2 * NSUB, NPAD)
    d0, d1 = dp[:NSUB], dp[NSUB:]
    h = _tc_layer(True, xp, p1[0], p1[1], d0, d1,
                  W_self1_0, W_self1_1, W_neigh1_0, W_neigh1_1,
                  b1_0.reshape(1, D), b1_1.reshape(1, D))
    p2, _ = _sc_agg(h, edges, zacc, zdeg)
    p2 = p2.reshape(2, NPAD, D)
    out = _tc_layer(False, h, p2[0], p2[1], d0, d1,
                    W_self2_0, W_self2_1, W_neigh2_0, W_neigh2_1,
                    b2_0.reshape(1, D), b2_1.reshape(1, D))
    return out[:N]


# speedup vs baseline: 9.3845x; 1.1841x over previous
import functools

import jax
import jax.numpy as jnp
from jax import lax
from jax.experimental import pallas as pl
from jax.experimental.pallas import tpu as pltpu
from jax.experimental.pallas import tpu_sc as plsc

N = 10000
NPAD = 10240
E = 320000
D = 128
NSUB = 16
ROWS_PER_TILE = NPAD // NSUB
EDGES_PER_TILE = E // NSUB
CHUNK = 128
NCHUNK = EDGES_PER_TILE // CHUNK
TAIL = EDGES_PER_TILE - NCHUNK * CHUNK
N_HOP = ROWS_PER_TILE // CHUNK
DEG_W = 16


def _sc_agg_body(x_hbm, edges_hbm, zacc_hbm, zdeg_hbm,
                 p_hbm, dpart_hbm,
                 acc, sp, dp, dst_v, tsrc_v, tdst_v, rows3, degacc_v,
                 gs, si, di):
    c = lax.axis_index("c")
    s = lax.axis_index("s")

    row0 = s * ROWS_PER_TILE
    pltpu.sync_copy(zacc_hbm, rows3.at[0])
    pltpu.sync_copy(zdeg_hbm, degacc_v)
    for j in range(N_HOP):
        pltpu.sync_copy(rows3.at[0], acc.at[pl.ds(row0 + j * CHUNK, CHUNK)])
    plsc.subcore_barrier()

    base = s * EDGES_PER_TILE
    rel0 = c * (2 * E)

    def src_off(i):
        return rel0 + base + i * CHUNK

    def dst_off(i):
        return rel0 + E + base + i * CHUNK

    pltpu.sync_copy(edges_hbm.at[pl.ds(src_off(NCHUNK), TAIL)], tsrc_v)
    pltpu.sync_copy(edges_hbm.at[pl.ds(dst_off(NCHUNK), TAIL)], tdst_v)
    pltpu.async_copy(x_hbm.at[tsrc_v], rows3.at[0, pl.ds(0, TAIL)], gs).wait()
    for k in range(TAIL // 16):
        tidx16 = tdst_v[pl.ds(k * 16, 16)]
        plsc.addupdate_scatter(degacc_v, [tidx16],
                               jnp.full((16,), 1.0, jnp.float32))
    pltpu.sync_copy(rows3.at[0, pl.ds(0, TAIL)], acc.at[tdst_v], add=True)

    pltpu.sync_copy(edges_hbm.at[pl.ds(src_off(0), CHUNK)], sp.at[0])
    pltpu.sync_copy(edges_hbm.at[pl.ds(dst_off(0), CHUNK)], dp.at[0])
    pltpu.async_copy(x_hbm.at[sp.at[0]], rows3.at[0], gs)
    pltpu.async_copy(edges_hbm.at[pl.ds(src_off(1), CHUNK)], sp.at[1], si)
    pltpu.async_copy(edges_hbm.at[pl.ds(dst_off(1), CHUNK)], dp.at[1], di)

    def step(i, carry):
        p = lax.rem(i, 2)
        q = 1 - p
        pltpu.make_async_copy(x_hbm.at[sp.at[p]], rows3.at[p], gs).wait()

        @pl.when(i < NCHUNK - 1)
        def _():
            pltpu.make_async_copy(
                edges_hbm.at[pl.ds(src_off(i + 1), CHUNK)], sp.at[q],
                si).wait()
            pltpu.async_copy(x_hbm.at[sp.at[q]], rows3.at[q], gs)

        @pl.when(i < NCHUNK - 2)
        def _():
            pltpu.async_copy(
                edges_hbm.at[pl.ds(src_off(i + 2), CHUNK)], sp.at[p], si)

        @pl.when(i > 0)
        def _():
            pltpu.make_async_copy(
                edges_hbm.at[pl.ds(dst_off(i), CHUNK)], dp.at[p], di).wait()

        @pl.when(i < NCHUNK - 1)
        def _():
            pltpu.async_copy(
                edges_hbm.at[pl.ds(dst_off(i + 1), CHUNK)], dp.at[q], di)
        for k in range(CHUNK // 16):
            idx16 = dp[p, pl.ds(k * 16, 16)]
            dst_v[pl.ds(k * 16, 16)] = idx16
            plsc.addupdate_scatter(degacc_v, [idx16],
                                   jnp.full((16,), 1.0, jnp.float32))
        pltpu.sync_copy(rows3.at[p], acc.at[dst_v], add=True)
        return carry

    lax.fori_loop(0, NCHUNK, step, 0)
    pltpu.sync_copy(degacc_v,
                    dpart_hbm.at[pl.ds((c * NSUB + s) * NPAD, NPAD)])
    plsc.subcore_barrier()

    out0 = c * NPAD + row0
    for j in range(N_HOP):
        pltpu.sync_copy(acc.at[pl.ds(row0 + j * CHUNK, CHUNK)], rows3.at[0])
        pltpu.sync_copy(rows3.at[0], p_hbm.at[pl.ds(out0 + j * CHUNK, CHUNK)])


_sc_agg = pl.kernel(
    _sc_agg_body,
    out_type=(
        jax.ShapeDtypeStruct((2 * NPAD, D), jnp.float32),
        jax.ShapeDtypeStruct((2 * NSUB * NPAD,), jnp.float32),
    ),
    mesh=plsc.VectorSubcoreMesh(core_axis_name="c", subcore_axis_name="s",
                                num_cores=2, num_subcores=NSUB),
    compiler_params=pltpu.CompilerParams(needs_layout_passes=False),
    scratch_types=[
        pltpu.VMEM_SHARED((NPAD, D), jnp.float32),
        pltpu.VMEM((2, CHUNK), jnp.int32),
        pltpu.VMEM((2, CHUNK), jnp.int32),
        pltpu.VMEM((CHUNK,), jnp.int32),
        pltpu.VMEM((TAIL,), jnp.int32),
        pltpu.VMEM((TAIL,), jnp.int32),
        pltpu.VMEM((2, CHUNK, D), jnp.float32),
        pltpu.VMEM((NPAD,), jnp.float32),
        pltpu.SemaphoreType.DMA,
        pltpu.SemaphoreType.DMA,
        pltpu.SemaphoreType.DMA,
    ],
)


def _tc_layer_body(x_ref, p0_ref, p1_ref, d0_ref, d1_ref,
                   ws0_ref, ws1_ref, wn0_ref, wn1_ref, b0_ref, b1_ref,
                   out_ref, *, relu):
    ones16 = jnp.ones((NSUB, 1), jnp.float32)
    dn = (((0,), (0,)), ((), ()))
    d0 = lax.dot_general(d0_ref[...], ones16, dn,
                         preferred_element_type=jnp.float32)
    d1 = lax.dot_general(d1_ref[...], ones16, dn,
                         preferred_element_type=jnp.float32)
    r0 = 1.0 / jnp.maximum(d0, 1.0)
    r1 = 1.0 / jnp.maximum(d1, 1.0)
    ws = ws0_ref[...] + ws1_ref[...]
    acc = jnp.dot(x_ref[...], ws, preferred_element_type=jnp.float32)
    acc += jnp.dot(p0_ref[...] * r0, wn0_ref[...],
                   preferred_element_type=jnp.float32)
    acc += jnp.dot(p1_ref[...] * r1, wn1_ref[...],
                   preferred_element_type=jnp.float32)
    acc += b0_ref[...] + b1_ref[...]
    out_ref[...] = jnp.maximum(acc, 0.0) if relu else acc


def _tc_layer(relu, x, p0, p1, d0, d1, ws0, ws1, wn0, wn1, b0, b1):
    blk = 1024
    grid = (NPAD // blk,)
    row_spec = pl.BlockSpec((blk, D), lambda i: (i, 0))
    deg_spec = pl.BlockSpec((NSUB, blk), lambda i: (0, i))
    w_spec = pl.BlockSpec((D, D), lambda i: (0, 0))
    b_spec = pl.BlockSpec((1, D), lambda i: (0, 0))
    return pl.pallas_call(
        functools.partial(_tc_layer_body, relu=relu),
        grid=grid,
        in_specs=[row_spec, row_spec, row_spec, deg_spec, deg_spec,
                  w_spec, w_spec, w_spec, w_spec, b_spec, b_spec],
        out_specs=row_spec,
        out_shape=jax.ShapeDtypeStruct((NPAD, D), jnp.float32),
    )(x, p0, p1, d0, d1, ws0, ws1, wn0, wn1, b0, b1)


def kernel(x, edge_index_rel0, edge_index_rel1,
           W_self1_0, W_neigh1_0, b1_0, W_self1_1, W_neigh1_1, b1_1,
           W_self2_0, W_neigh2_0, b2_0, W_self2_1, W_neigh2_1, b2_1):
    edges = jnp.concatenate([edge_index_rel0.reshape(-1),
                             edge_index_rel1.reshape(-1)])
    zacc = jnp.zeros((CHUNK, D), jnp.float32)
    zdeg = jnp.zeros((NPAD,), jnp.float32)
    xp = jnp.pad(x, ((0, NPAD - N), (0, 0)))

    p1, dpart = _sc_agg(xp, edges, zacc, zdeg)
    p1 = p1.reshape(2, NPAD, D)
    dp = dpart.reshape(2 * NSUB, NPAD)
    d0, d1 = dp[:NSUB], dp[NSUB:]
    h = _tc_layer(True, xp, p1[0], p1[1], d0, d1,
                  W_self1_0, W_self1_1, W_neigh1_0, W_neigh1_1,
                  b1_0.reshape(1, D), b1_1.reshape(1, D))
    p2, _ = _sc_agg(h, edges, zacc, zdeg)
    p2 = p2.reshape(2, NPAD, D)
    out = _tc_layer(False, h, p2[0], p2[1], d0, d1,
                    W_self2_0, W_self2_1, W_neigh2_0, W_neigh2_1,
                    b2_0.reshape(1, D), b2_1.reshape(1, D))
    return out[:N]


# speedup vs baseline: 9.4193x; 1.0037x over previous
import functools

import jax
import jax.numpy as jnp
from jax import lax
from jax.experimental import pallas as pl
from jax.experimental.pallas import tpu as pltpu
from jax.experimental.pallas import tpu_sc as plsc

N = 10000
NPAD = 10240
E = 320000
D = 128
NSUB = 16
ROWS_PER_TILE = NPAD // NSUB
EDGES_PER_TILE = E // NSUB
CHUNK = 128
NCHUNK = EDGES_PER_TILE // CHUNK
TAIL = EDGES_PER_TILE - NCHUNK * CHUNK
N_HOP = ROWS_PER_TILE // CHUNK
DEG_W = 16


def _sc_agg_body(x_hbm, edges_hbm, zacc_hbm, zdeg_hbm,
                 p_hbm, dpart_hbm,
                 acc, sp, dp, dst_v, tsrc_v, tdst_v, rows3, degacc_v,
                 gs, si, di, *, with_deg):
    c = lax.axis_index("c")
    s = lax.axis_index("s")

    row0 = s * ROWS_PER_TILE
    pltpu.sync_copy(zacc_hbm, rows3.at[0])
    pltpu.sync_copy(zdeg_hbm, degacc_v)
    for j in range(N_HOP):
        pltpu.sync_copy(rows3.at[0], acc.at[pl.ds(row0 + j * CHUNK, CHUNK)])
    plsc.subcore_barrier()

    base = s * EDGES_PER_TILE
    rel0 = c * (2 * E)

    def src_off(i):
        return rel0 + base + i * CHUNK

    def dst_off(i):
        return rel0 + E + base + i * CHUNK

    pltpu.sync_copy(edges_hbm.at[pl.ds(src_off(NCHUNK), TAIL)], tsrc_v)
    pltpu.sync_copy(edges_hbm.at[pl.ds(dst_off(NCHUNK), TAIL)], tdst_v)
    pltpu.async_copy(x_hbm.at[tsrc_v], rows3.at[0, pl.ds(0, TAIL)], gs).wait()
    if with_deg:
        for k in range(TAIL // 16):
            tidx16 = tdst_v[pl.ds(k * 16, 16)]
            plsc.addupdate_scatter(degacc_v, [tidx16],
                                   jnp.full((16,), 1.0, jnp.float32))
    pltpu.sync_copy(rows3.at[0, pl.ds(0, TAIL)], acc.at[tdst_v], add=True)

    pltpu.sync_copy(edges_hbm.at[pl.ds(src_off(0), CHUNK)], sp.at[0])
    pltpu.sync_copy(edges_hbm.at[pl.ds(dst_off(0), CHUNK)], dp.at[0])
    pltpu.async_copy(x_hbm.at[sp.at[0]], rows3.at[0], gs)
    pltpu.async_copy(edges_hbm.at[pl.ds(src_off(1), CHUNK)], sp.at[1], si)
    pltpu.async_copy(edges_hbm.at[pl.ds(dst_off(1), CHUNK)], dp.at[1], di)

    def step(i, carry):
        p = lax.rem(i, 2)
        q = 1 - p
        pltpu.make_async_copy(x_hbm.at[sp.at[p]], rows3.at[p], gs).wait()

        @pl.when(i < NCHUNK - 1)
        def _():
            pltpu.make_async_copy(
                edges_hbm.at[pl.ds(src_off(i + 1), CHUNK)], sp.at[q],
                si).wait()
            pltpu.async_copy(x_hbm.at[sp.at[q]], rows3.at[q], gs)

        @pl.when(i < NCHUNK - 2)
        def _():
            pltpu.async_copy(
                edges_hbm.at[pl.ds(src_off(i + 2), CHUNK)], sp.at[p], si)

        @pl.when(i > 0)
        def _():
            pltpu.make_async_copy(
                edges_hbm.at[pl.ds(dst_off(i), CHUNK)], dp.at[p], di).wait()

        @pl.when(i < NCHUNK - 1)
        def _():
            pltpu.async_copy(
                edges_hbm.at[pl.ds(dst_off(i + 1), CHUNK)], dp.at[q], di)
        for k in range(CHUNK // 16):
            idx16 = dp[p, pl.ds(k * 16, 16)]
            dst_v[pl.ds(k * 16, 16)] = idx16
            if with_deg:
                plsc.addupdate_scatter(degacc_v, [idx16],
                                       jnp.full((16,), 1.0, jnp.float32))
        pltpu.sync_copy(rows3.at[p], acc.at[dst_v], add=True)
        return carry

    lax.fori_loop(0, NCHUNK, step, 0)
    if with_deg:
        pltpu.sync_copy(degacc_v,
                        dpart_hbm.at[pl.ds((c * NSUB + s) * NPAD, NPAD)])
    plsc.subcore_barrier()

    out0 = c * NPAD + row0
    for j in range(N_HOP):
        pltpu.sync_copy(acc.at[pl.ds(row0 + j * CHUNK, CHUNK)], rows3.at[0])
        pltpu.sync_copy(rows3.at[0], p_hbm.at[pl.ds(out0 + j * CHUNK, CHUNK)])


def _make_sc_agg(with_deg):
  return pl.kernel(
    functools.partial(_sc_agg_body, with_deg=with_deg),
    out_type=(
        jax.ShapeDtypeStruct((2 * NPAD, D), jnp.float32),
        jax.ShapeDtypeStruct((2 * NSUB * NPAD,), jnp.float32),
    ),
    mesh=plsc.VectorSubcoreMesh(core_axis_name="c", subcore_axis_name="s",
                                num_cores=2, num_subcores=NSUB),
    compiler_params=pltpu.CompilerParams(needs_layout_passes=False),
    scratch_types=[
        pltpu.VMEM_SHARED((NPAD, D), jnp.float32),
        pltpu.VMEM((2, CHUNK), jnp.int32),
        pltpu.VMEM((2, CHUNK), jnp.int32),
        pltpu.VMEM((CHUNK,), jnp.int32),
        pltpu.VMEM((TAIL,), jnp.int32),
        pltpu.VMEM((TAIL,), jnp.int32),
        pltpu.VMEM((2, CHUNK, D), jnp.float32),
        pltpu.VMEM((NPAD,), jnp.float32),
        pltpu.SemaphoreType.DMA,
        pltpu.SemaphoreType.DMA,
        pltpu.SemaphoreType.DMA,
    ],
  )


_sc_agg = _make_sc_agg(True)
_sc_agg_nodeg = _make_sc_agg(False)


def _tc_layer_body(x_ref, p0_ref, p1_ref, d0_ref, d1_ref,
                   ws0_ref, ws1_ref, wn0_ref, wn1_ref, b0_ref, b1_ref,
                   out_ref, *, relu):
    ones16 = jnp.ones((NSUB, 1), jnp.float32)
    dn = (((0,), (0,)), ((), ()))
    d0 = lax.dot_general(d0_ref[...], ones16, dn,
                         preferred_element_type=jnp.float32)
    d1 = lax.dot_general(d1_ref[...], ones16, dn,
                         preferred_element_type=jnp.float32)
    r0 = 1.0 / jnp.maximum(d0, 1.0)
    r1 = 1.0 / jnp.maximum(d1, 1.0)
    ws = ws0_ref[...] + ws1_ref[...]
    acc = jnp.dot(x_ref[...], ws, preferred_element_type=jnp.float32)
    acc += jnp.dot(p0_ref[...] * r0, wn0_ref[...],
                   preferred_element_type=jnp.float32)
    acc += jnp.dot(p1_ref[...] * r1, wn1_ref[...],
                   preferred_element_type=jnp.float32)
    acc += b0_ref[...] + b1_ref[...]
    out_ref[...] = jnp.maximum(acc, 0.0) if relu else acc


def _tc_layer(relu, x, p0, p1, d0, d1, ws0, ws1, wn0, wn1, b0, b1):
    blk = 1024
    grid = (NPAD // blk,)
    row_spec = pl.BlockSpec((blk, D), lambda i: (i, 0))
    deg_spec = pl.BlockSpec((NSUB, blk), lambda i: (0, i))
    w_spec = pl.BlockSpec((D, D), lambda i: (0, 0))
    b_spec = pl.BlockSpec((1, D), lambda i: (0, 0))
    return pl.pallas_call(
        functools.partial(_tc_layer_body, relu=relu),
        grid=grid,
        in_specs=[row_spec, row_spec, row_spec, deg_spec, deg_spec,
                  w_spec, w_spec, w_spec, w_spec, b_spec, b_spec],
        out_specs=row_spec,
        out_shape=jax.ShapeDtypeStruct((NPAD, D), jnp.float32),
    )(x, p0, p1, d0, d1, ws0, ws1, wn0, wn1, b0, b1)


def kernel(x, edge_index_rel0, edge_index_rel1,
           W_self1_0, W_neigh1_0, b1_0, W_self1_1, W_neigh1_1, b1_1,
           W_self2_0, W_neigh2_0, b2_0, W_self2_1, W_neigh2_1, b2_1):
    edges = jnp.concatenate([edge_index_rel0.reshape(-1),
                             edge_index_rel1.reshape(-1)])
    zacc = jnp.zeros((CHUNK, D), jnp.float32)
    zdeg = jnp.zeros((NPAD,), jnp.float32)
    xp = jnp.pad(x, ((0, NPAD - N), (0, 0)))

    p1, dpart = _sc_agg(xp, edges, zacc, zdeg)
    p1 = p1.reshape(2, NPAD, D)
    dp = dpart.reshape(2 * NSUB, NPAD)
    d0, d1 = dp[:NSUB], dp[NSUB:]
    h = _tc_layer(True, xp, p1[0], p1[1], d0, d1,
                  W_self1_0, W_self1_1, W_neigh1_0, W_neigh1_1,
                  b1_0.reshape(1, D), b1_1.reshape(1, D))
    p2, _ = _sc_agg(h, edges, zacc, zdeg)
    p2 = p2.reshape(2, NPAD, D)
    out = _tc_layer(False, h, p2[0], p2[1], d0, d1,
                    W_self2_0, W_self2_1, W_neigh2_0, W_neigh2_1,
                    b2_0.reshape(1, D), b2_1.reshape(1, D))
    return out[:N]
